# baseline (speedup 1.0000x reference)
import functools

import jax
import jax.numpy as jnp
from jax import lax
from jax.experimental import pallas as pl
from jax.experimental.pallas import tpu as pltpu
from jax.experimental.pallas import tpu_sc as plsc

B, LQ, D, H, L, P = 4, 2048, 512, 16, 3, 4
HD = D // H
LEVELS = ((64, 64, 0), (32, 32, 4096), (16, 16, 5120))
LV = 5376
QB = 256
NQR = B * LQ
NS_PER_ROW = H * L * P * 4
NS_IDX = H * L * P * 2
NC, NSUB, LANES = 2, 16, 16
NW = NC * NSUB
ROWS_PER_W = NQR // NW
E_ROWS = B * H * LV // 2


def _head_body(row0, x_ref, wq_ref, wref_ref, woff_ref, wattn_ref,
               idx_ref, wgt_ref):
    xq = x_ref[...]
    q = jnp.dot(xq, wq_ref[...], preferred_element_type=jnp.float32)
    rr = jnp.dot(q, wref_ref[...], preferred_element_type=jnp.float32)
    off = jnp.dot(q, woff_ref[...], preferred_element_type=jnp.float32)
    al = jnp.dot(q, wattn_ref[...], preferred_element_type=jnp.float32)

    rx = jax.nn.sigmoid(rr[:, 0:64])
    ry = jax.nn.sigmoid(rr[:, 64:128])

    m16 = al[:, 0:16]
    for s in range(1, 12):
        m16 = jnp.maximum(m16, al[:, s * 16:(s + 1) * 16])
    e_ls = []
    s16 = jnp.zeros_like(m16)
    for lev in range(L):
        m64 = jnp.concatenate([m16, m16, m16, m16], axis=1)
        e = jnp.exp(al[:, lev * 64:(lev + 1) * 64] - m64)
        e_ls.append(e)
        for p_ in range(4):
            s16 = s16 + e[:, p_ * 16:(p_ + 1) * 16]
    inv16 = 1.0 / s16
    inv64 = jnp.concatenate([inv16, inv16, inv16, inv16], axis=1)

    h_lane = lax.broadcasted_iota(jnp.int32, (QB, 64), 1) % 16
    bidx = (row0 + pl.program_id(0) * QB) // LQ

    for lev, (hl, wl, start) in enumerate(LEVELS):
        attn_l = e_ls[lev] * inv64
        ox = off[:, (lev * 2 + 0) * 64:(lev * 2 + 1) * 64]
        oy = off[:, (lev * 2 + 1) * 64:(lev * 2 + 2) * 64]
        ix = (rx + ox) * wl - 0.5
        iy = (ry + oy) * hl - 0.5
        x0 = jnp.floor(ix)
        y0 = jnp.floor(iy)
        fx = ix - x0
        fy = iy - y0
        vx0 = ((x0 >= 0) & (x0 <= wl - 1)).astype(jnp.float32)
        vx1 = ((x0 >= -1) & (x0 <= wl - 2)).astype(jnp.float32)
        vy0 = ((y0 >= 0) & (y0 <= hl - 1)).astype(jnp.float32)
        vy1 = ((y0 >= -1) & (y0 <= hl - 2)).astype(jnp.float32)
        x0c = jnp.clip(x0, -1, wl - 1).astype(jnp.int32)
        neg = (x0c < 0)
        cy0 = jnp.clip(y0, 0, hl - 1).astype(jnp.int32)
        cy1 = jnp.clip(y0 + 1, 0, hl - 1).astype(jnp.int32)
        cellb = (bidx * H) * LV + h_lane * LV + start
        wa_x0 = attn_l * (1.0 - fx) * vx0
        wa_x1 = attn_l * fx * vx1
        w0x = jnp.where(neg, wa_x1, wa_x0)
        w1x = jnp.where(neg, 0.0, wa_x1)
        wy = (1.0 - fy, fy)
        cys = (cy0, cy1)
        vys = (vy0, vy1)
        xoff = x0c + neg.astype(jnp.int32)
        for yc in range(2):
            s = cellb + cys[yc] * wl + xoff
            parity = s & 1
            row = (s >> 1) + parity * E_ROWS
            idx_ref[:, pl.ds((lev * 2 + yc) * 64, 64)] = row
            w0 = w0x * wy[yc] * vys[yc]
            w1 = w1x * wy[yc] * vys[yc]
            for p_ in range(4):
                dst = lev * 256 + yc * 128 + p_ * 32
                wgt_ref[:, pl.ds(dst, 16)] = w0[:, p_ * 16:(p_ + 1) * 16]
                wgt_ref[:, pl.ds(dst + 16, 16)] = w1[:, p_ * 16:(p_ + 1) * 16]


def _matmul_body(a_ref, w_ref, o_ref):
    o_ref[...] = jnp.dot(a_ref[...], w_ref[...],
                         preferred_element_type=jnp.float32
                         ).astype(o_ref.dtype)


def _pack_cells(o):
    lo = jnp.concatenate([o[:, h * HD:h * HD + 16] for h in range(H)], axis=1)
    hi = jnp.concatenate([o[:, h * HD + 16:(h + 1) * HD] for h in range(H)],
                         axis=1)
    lo_b = lax.bitcast_convert_type(lo.astype(jnp.bfloat16),
                                    jnp.uint16).astype(jnp.uint32)
    hi_b = lax.bitcast_convert_type(hi.astype(jnp.bfloat16),
                                    jnp.uint16).astype(jnp.uint32)
    return lax.bitcast_convert_type(
        lo_b | lax.shift_left(hi_b, jnp.uint32(16)), jnp.int32)


def _values_body(e_ref, o_ref, w_ref, t_ref):
    pe = _pack_cells(jnp.dot(e_ref[...], w_ref[...],
                             preferred_element_type=jnp.float32))
    po = _pack_cells(jnp.dot(o_ref[...], w_ref[...],
                             preferred_element_type=jnp.float32))
    pe_s = jnp.concatenate([pe[1:, :], pe[0:1, :]], axis=0)
    for h in range(H):
        lo, hi = h * 16, (h + 1) * 16
        t_ref[0, 0, h, :, pl.ds(0, 16)] = pe[:, lo:hi]
        t_ref[0, 0, h, :, pl.ds(16, 16)] = po[:, lo:hi]
        t_ref[1, 0, h, :, pl.ds(0, 16)] = po[:, lo:hi]
        t_ref[1, 0, h, :, pl.ds(16, 16)] = pe_s[:, lo:hi]


def _splat(vec, h):
    idx = jnp.zeros((LANES,), jnp.int32) + h
    return lax.gather(
        vec, idx[:, None],
        lax.GatherDimensionNumbers(
            offset_dims=(), collapsed_slice_dims=(0,), start_index_map=(0,)),
        slice_sizes=(1,),
        mode=lax.GatherScatterMode.PROMISE_IN_BOUNDS)


def _sc_sample_body(rows_per_w, table_hbm, idx_hbm, wgt_hbm, out_hbm,
                    idx_v, wgt_v, rows_v, out_v, gsem, iwsem):
    wid = lax.axis_index("s") * NC + lax.axis_index("c")
    base = wid * rows_per_w
    nj = L * P * 2

    def issue_iw(r, slot):
        pltpu.async_copy(idx_hbm.at[base + r], idx_v.at[slot], iwsem.at[slot])
        pltpu.async_copy(wgt_hbm.at[base + r], wgt_v.at[slot], iwsem.at[slot])

    def wait_iw(slot):
        pltpu.make_async_copy(idx_hbm.at[0], idx_v.at[slot],
                              iwsem.at[slot]).wait()
        pltpu.make_async_copy(wgt_hbm.at[0], wgt_v.at[slot],
                              iwsem.at[slot]).wait()

    def issue_gathers(slot):
        pltpu.async_copy(table_hbm.at[idx_v.at[slot]],
                         rows_v.at[slot], gsem.at[slot])

    def wait_gathers(slot):
        pltpu.make_async_copy(table_hbm.at[pl.ds(0, NS_IDX)],
                              rows_v.at[slot], gsem.at[slot]).wait()

    issue_iw(0, 0)
    issue_iw(1, 1)
    wait_iw(0)
    issue_gathers(0)

    def row_body(r, carry):
        slot = r % 3
        wait_gathers(slot)

        @pl.when(r + 2 < rows_per_w)
        def _():
            issue_iw(r + 2, (r + 2) % 3)

        @pl.when(r + 1 < rows_per_w)
        def _():
            nslot = (r + 1) % 3
            wait_iw(nslot)
            issue_gathers(nslot)

        for hbase in (0, 8):
            def j_body(j, accs, hbase=hbase):
                wv0 = wgt_v[slot, pl.ds(j * 32, 16)]
                wv1 = wgt_v[slot, pl.ds(j * 32 + 16, 16)]
                new = []
                for hh in range(8):
                    h = hbase + hh
                    wsp0 = _splat(wv0, h)
                    wsp1 = _splat(wv1, h)
                    c0 = rows_v[slot, j * 16 + h, pl.ds(0, 16)]
                    c1 = rows_v[slot, j * 16 + h, pl.ds(16, 16)]
                    a = accs[2 * hh] + wsp0 * lax.bitcast_convert_type(
                        lax.shift_left(c0, 16), jnp.float32)
                    a = a + wsp1 * lax.bitcast_convert_type(
                        lax.shift_left(c1, 16), jnp.float32)
                    b = accs[2 * hh + 1] + wsp0 * lax.bitcast_convert_type(
                        c0, jnp.float32)
                    b = b + wsp1 * lax.bitcast_convert_type(c1, jnp.float32)
                    new.append(a)
                    new.append(b)
                return tuple(new)

            accs = lax.fori_loop(
                0, nj, j_body,
                tuple(jnp.zeros((LANES,), jnp.float32) for _ in range(16)))
            for hh in range(8):
                h = hbase + hh
                out_v[pl.ds(h * HD, 16)] = accs[2 * hh]
                out_v[pl.ds(h * HD + 16, 16)] = accs[2 * hh + 1]
        pltpu.sync_copy(out_v, out_hbm.at[base + r])
        return carry

    lax.fori_loop(0, rows_per_w, row_body, 0)


def kernel(x, encoder_input, spatial_shapes, Wq, Wref, Woff, Wattn, Wv, Wout):
    del spatial_shapes
    x2 = x.reshape(NQR, D)
    enc2 = encoder_input.reshape(B * LV, D)

    woff_p = Woff.reshape(D, H, L, P, 2).transpose(0, 2, 4, 3, 1).reshape(D, 384)
    wattn_p = Wattn.reshape(D, H, L * P).transpose(0, 2, 1).reshape(D, 192)
    wref_p = jnp.concatenate([jnp.tile(Wref[:, 0:1], (1, 64)),
                              jnp.tile(Wref[:, 1:2], (1, 64))], axis=1)

    enc3 = encoder_input.reshape(B, LV, D)
    enc_e = enc3[:, 0::2, :].reshape(B * LV // 2, D)
    enc_o = enc3[:, 1::2, :].reshape(B * LV // 2, D)
    lvb = LV // 2 // 128

    vt = pl.pallas_call(
        _values_body,
        grid=(B, lvb),
        in_specs=[
            pl.BlockSpec((128, D), lambda b, l: (b * lvb + l, 0)),
            pl.BlockSpec((128, D), lambda b, l: (b * lvb + l, 0)),
            pl.BlockSpec((D, D), lambda b, l: (0, 0)),
        ],
        out_specs=pl.BlockSpec((2, 1, H, 128, 32),
                               lambda b, l: (0, b, 0, l, 0)),
        out_shape=jax.ShapeDtypeStruct((2, B, H, LV // 2, 32), jnp.int32),
    )(enc_e, enc_o, Wv)

    table = vt.reshape(2 * E_ROWS, 32)
    mesh = plsc.VectorSubcoreMesh(core_axis_name="c", subcore_axis_name="s")

    nchunk = 4
    crows = NQR // nchunk
    n_blocks = crows // QB
    outs = []
    for ci in range(nchunk):
        row0 = ci * crows
        idx, wgt = pl.pallas_call(
            functools.partial(_head_body, row0),
            grid=(n_blocks,),
            in_specs=[
                pl.BlockSpec((QB, D), lambda i: (i, 0)),
                pl.BlockSpec((D, D), lambda i: (0, 0)),
                pl.BlockSpec((D, 128), lambda i: (0, 0)),
                pl.BlockSpec((D, 384), lambda i: (0, 0)),
                pl.BlockSpec((D, 192), lambda i: (0, 0)),
            ],
            out_specs=[
                pl.BlockSpec((QB, NS_IDX), lambda i: (i, 0)),
                pl.BlockSpec((QB, NS_PER_ROW), lambda i: (i, 0)),
            ],
            out_shape=[
                jax.ShapeDtypeStruct((crows, NS_IDX), jnp.int32),
                jax.ShapeDtypeStruct((crows, NS_PER_ROW), jnp.float32),
            ],
        )(x2[row0:row0 + crows], Wq, wref_p, woff_p, wattn_p)

        sampled = pl.kernel(
            functools.partial(_sc_sample_body, crows // NW),
            mesh=mesh,
            compiler_params=pltpu.CompilerParams(use_tc_tiling_on_sc=False),
            out_type=jax.ShapeDtypeStruct((crows, D), jnp.float32),
            scratch_types=[
                pltpu.VMEM((3, NS_IDX), jnp.int32),
                pltpu.VMEM((3, NS_PER_ROW), jnp.float32),
                pltpu.VMEM((3, NS_IDX, 32), jnp.int32),
                pltpu.VMEM((D,), jnp.float32),
                pltpu.SemaphoreType.DMA((3,)),
                pltpu.SemaphoreType.DMA((3,)),
            ],
        )(table, idx, wgt)

        out = pl.pallas_call(
            _matmul_body,
            grid=(n_blocks,),
            in_specs=[
                pl.BlockSpec((QB, D), lambda i: (i, 0)),
                pl.BlockSpec((D, D), lambda i: (0, 0)),
            ],
            out_specs=pl.BlockSpec((QB, D), lambda i: (i, 0)),
            out_shape=jax.ShapeDtypeStruct((crows, D), jnp.float32),
        )(sampled, Wout)
        outs.append(out)

    return jnp.concatenate(outs, axis=0).reshape(B, LQ, D)


# speedup vs baseline: 1.1651x; 1.1651x over previous
import functools

import jax
import jax.numpy as jnp
from jax import lax
from jax.experimental import pallas as pl
from jax.experimental.pallas import tpu as pltpu
from jax.experimental.pallas import tpu_sc as plsc

B, LQ, D, H, L, P = 4, 2048, 512, 16, 3, 4
HD = D // H
LEVELS = ((64, 64, 0), (32, 32, 4096), (16, 16, 5120))
LV = 5376
QB = 256
NQR = B * LQ
NS_PER_ROW = H * L * P * 4
NS_IDX = H * L * P * 2
NC, NSUB, LANES = 2, 16, 16
NW = NC * NSUB
ROWS_PER_W = NQR // NW
E_ROWS = B * H * LV // 2


def _head_body(row0, x_ref, wq_ref, wref_ref, woff_ref, wattn_ref,
               idx_ref, wgt_ref):
    xq = x_ref[...]
    q = jnp.dot(xq, wq_ref[...], preferred_element_type=jnp.float32)
    rr = jnp.dot(q, wref_ref[...], preferred_element_type=jnp.float32)
    off = jnp.dot(q, woff_ref[...], preferred_element_type=jnp.float32)
    al = jnp.dot(q, wattn_ref[...], preferred_element_type=jnp.float32)

    rx = jax.nn.sigmoid(rr[:, 0:64])
    ry = jax.nn.sigmoid(rr[:, 64:128])

    m16 = al[:, 0:16]
    for s in range(1, 12):
        m16 = jnp.maximum(m16, al[:, s * 16:(s + 1) * 16])
    e_ls = []
    s16 = jnp.zeros_like(m16)
    for lev in range(L):
        m64 = jnp.concatenate([m16, m16, m16, m16], axis=1)
        e = jnp.exp(al[:, lev * 64:(lev + 1) * 64] - m64)
        e_ls.append(e)
        for p_ in range(4):
            s16 = s16 + e[:, p_ * 16:(p_ + 1) * 16]
    inv16 = 1.0 / s16
    inv64 = jnp.concatenate([inv16, inv16, inv16, inv16], axis=1)

    h_lane = lax.broadcasted_iota(jnp.int32, (QB, 64), 1) % 16
    bidx = (row0 + pl.program_id(0) * QB) // LQ

    for lev, (hl, wl, start) in enumerate(LEVELS):
        attn_l = e_ls[lev] * inv64
        ox = off[:, (lev * 2 + 0) * 64:(lev * 2 + 1) * 64]
        oy = off[:, (lev * 2 + 1) * 64:(lev * 2 + 2) * 64]
        ix = (rx + ox) * wl - 0.5
        iy = (ry + oy) * hl - 0.5
        x0 = jnp.floor(ix)
        y0 = jnp.floor(iy)
        fx = ix - x0
        fy = iy - y0
        vx0 = ((x0 >= 0) & (x0 <= wl - 1)).astype(jnp.float32)
        vx1 = ((x0 >= -1) & (x0 <= wl - 2)).astype(jnp.float32)
        vy0 = ((y0 >= 0) & (y0 <= hl - 1)).astype(jnp.float32)
        vy1 = ((y0 >= -1) & (y0 <= hl - 2)).astype(jnp.float32)
        x0c = jnp.clip(x0, -1, wl - 1).astype(jnp.int32)
        neg = (x0c < 0)
        cy0 = jnp.clip(y0, 0, hl - 1).astype(jnp.int32)
        cy1 = jnp.clip(y0 + 1, 0, hl - 1).astype(jnp.int32)
        cellb = (bidx * H) * LV + h_lane * LV + start
        wa_x0 = attn_l * (1.0 - fx) * vx0
        wa_x1 = attn_l * fx * vx1
        w0x = jnp.where(neg, wa_x1, wa_x0)
        w1x = jnp.where(neg, 0.0, wa_x1)
        wy = (1.0 - fy, fy)
        cys = (cy0, cy1)
        vys = (vy0, vy1)
        xoff = x0c + neg.astype(jnp.int32)
        for yc in range(2):
            s = cellb + cys[yc] * wl + xoff
            parity = s & 1
            row = lax.shift_left(s >> 1, 2) + parity
            idx_ref[:, pl.ds((lev * 2 + yc) * 64, 64)] = row
            w0 = w0x * wy[yc] * vys[yc]
            w1 = w1x * wy[yc] * vys[yc]
            for p_ in range(4):
                dst = lev * 256 + yc * 128 + p_ * 32
                wgt_ref[:, pl.ds(dst, 16)] = w0[:, p_ * 16:(p_ + 1) * 16]
                wgt_ref[:, pl.ds(dst + 16, 16)] = w1[:, p_ * 16:(p_ + 1) * 16]


def _matmul_body(a_ref, w_ref, o_ref):
    o_ref[...] = jnp.dot(a_ref[...], w_ref[...],
                         preferred_element_type=jnp.float32
                         ).astype(o_ref.dtype)


def _pack_cells(o):
    lo = jnp.concatenate([o[:, h * HD:h * HD + 16] for h in range(H)], axis=1)
    hi = jnp.concatenate([o[:, h * HD + 16:(h + 1) * HD] for h in range(H)],
                         axis=1)
    lo_b = lax.bitcast_convert_type(lo.astype(jnp.bfloat16),
                                    jnp.uint16).astype(jnp.uint32)
    hi_b = lax.bitcast_convert_type(hi.astype(jnp.bfloat16),
                                    jnp.uint16).astype(jnp.uint32)
    return lax.bitcast_convert_type(
        lo_b | lax.shift_left(hi_b, jnp.uint32(16)), jnp.int32)


def _values_body(e_ref, o_ref, w_ref, t_ref):
    pe = _pack_cells(jnp.dot(e_ref[...], w_ref[...],
                             preferred_element_type=jnp.float32))
    po = _pack_cells(jnp.dot(o_ref[...], w_ref[...],
                             preferred_element_type=jnp.float32))
    pe_s = jnp.concatenate([pe[1:, :], pe[0:1, :]], axis=0)
    for h in range(H):
        lo, hi = h * 16, (h + 1) * 16
        t_ref[0, h] = jnp.concatenate(
            [pe[:, lo:hi], po[:, lo:hi], po[:, lo:hi], pe_s[:, lo:hi]] * 2,
            axis=1)


def _splat(vec, h):
    idx = jnp.zeros((LANES,), jnp.int32) + h
    return lax.gather(
        vec, idx[:, None],
        lax.GatherDimensionNumbers(
            offset_dims=(), collapsed_slice_dims=(0,), start_index_map=(0,)),
        slice_sizes=(1,),
        mode=lax.GatherScatterMode.PROMISE_IN_BOUNDS)


def _sc_sample_body(rows_per_w, table_hbm, idx_hbm, wgt_hbm, out_hbm,
                    idx_v, wgt_v, rows_v, out_v, gsem, iwsem):
    wid = lax.axis_index("s") * NC + lax.axis_index("c")
    base = wid * rows_per_w
    nj = L * P * 2

    def issue_iw(r, slot):
        pltpu.async_copy(idx_hbm.at[base + r], idx_v.at[slot], iwsem.at[slot])
        pltpu.async_copy(wgt_hbm.at[base + r], wgt_v.at[slot], iwsem.at[slot])

    def wait_iw(slot):
        pltpu.make_async_copy(idx_hbm.at[0], idx_v.at[slot],
                              iwsem.at[slot]).wait()
        pltpu.make_async_copy(wgt_hbm.at[0], wgt_v.at[slot],
                              iwsem.at[slot]).wait()

    def issue_gathers(slot):
        pltpu.async_copy(table_hbm.at[idx_v.at[slot]],
                         rows_v.at[slot], gsem.at[slot])

    def wait_gathers(slot):
        pltpu.make_async_copy(table_hbm.at[pl.ds(0, NS_IDX)],
                              rows_v.at[slot], gsem.at[slot]).wait()

    issue_iw(0, 0)
    issue_iw(1, 1)
    wait_iw(0)
    issue_gathers(0)

    def row_body(r, carry):
        slot = r % 3
        wait_gathers(slot)

        @pl.when(r + 2 < rows_per_w)
        def _():
            issue_iw(r + 2, (r + 2) % 3)

        @pl.when(r + 1 < rows_per_w)
        def _():
            nslot = (r + 1) % 3
            wait_iw(nslot)
            issue_gathers(nslot)

        for hbase in (0, 8):
            def j_body(j, accs, hbase=hbase):
                wv0 = wgt_v[slot, pl.ds(j * 32, 16)]
                wv1 = wgt_v[slot, pl.ds(j * 32 + 16, 16)]
                new = []
                for hh in range(8):
                    h = hbase + hh
                    wsp0 = _splat(wv0, h)
                    wsp1 = _splat(wv1, h)
                    c0 = rows_v[slot, j * 16 + h, pl.ds(0, 16)]
                    c1 = rows_v[slot, j * 16 + h, pl.ds(16, 16)]
                    a = accs[2 * hh] + wsp0 * lax.bitcast_convert_type(
                        lax.shift_left(c0, 16), jnp.float32)
                    a = a + wsp1 * lax.bitcast_convert_type(
                        lax.shift_left(c1, 16), jnp.float32)
                    b = accs[2 * hh + 1] + wsp0 * lax.bitcast_convert_type(
                        c0, jnp.float32)
                    b = b + wsp1 * lax.bitcast_convert_type(c1, jnp.float32)
                    new.append(a)
                    new.append(b)
                return tuple(new)

            accs = lax.fori_loop(
                0, nj, j_body,
                tuple(jnp.zeros((LANES,), jnp.float32) for _ in range(16)))
            for hh in range(8):
                h = hbase + hh
                out_v[pl.ds(h * HD, 16)] = accs[2 * hh]
                out_v[pl.ds(h * HD + 16, 16)] = accs[2 * hh + 1]
        pltpu.sync_copy(out_v, out_hbm.at[base + r])
        return carry

    lax.fori_loop(0, rows_per_w, row_body, 0)


def kernel(x, encoder_input, spatial_shapes, Wq, Wref, Woff, Wattn, Wv, Wout):
    del spatial_shapes
    x2 = x.reshape(NQR, D)
    enc2 = encoder_input.reshape(B * LV, D)

    woff_p = Woff.reshape(D, H, L, P, 2).transpose(0, 2, 4, 3, 1).reshape(D, 384)
    wattn_p = Wattn.reshape(D, H, L * P).transpose(0, 2, 1).reshape(D, 192)
    wref_p = jnp.concatenate([jnp.tile(Wref[:, 0:1], (1, 64)),
                              jnp.tile(Wref[:, 1:2], (1, 64))], axis=1)

    enc3 = encoder_input.reshape(B, LV, D)
    enc_e = enc3[:, 0::2, :].reshape(B * LV // 2, D)
    enc_o = enc3[:, 1::2, :].reshape(B * LV // 2, D)
    lvb = LV // 2 // 128

    vt = pl.pallas_call(
        _values_body,
        grid=(B, lvb),
        in_specs=[
            pl.BlockSpec((128, D), lambda b, l: (b * lvb + l, 0)),
            pl.BlockSpec((128, D), lambda b, l: (b * lvb + l, 0)),
            pl.BlockSpec((D, D), lambda b, l: (0, 0)),
        ],
        out_specs=pl.BlockSpec((1, H, 128, 128),
                               lambda b, l: (b, 0, l, 0)),
        out_shape=jax.ShapeDtypeStruct((B, H, LV // 2, 128), jnp.int32),
    )(enc_e, enc_o, Wv)

    table = vt.reshape(B * H * (LV // 2) * 4, 32)
    mesh = plsc.VectorSubcoreMesh(core_axis_name="c", subcore_axis_name="s")

    nchunk = 4
    crows = NQR // nchunk
    n_blocks = crows // QB
    outs = []
    for ci in range(nchunk):
        row0 = ci * crows
        idx, wgt = pl.pallas_call(
            functools.partial(_head_body, row0),
            grid=(n_blocks,),
            in_specs=[
                pl.BlockSpec((QB, D), lambda i: (i, 0)),
                pl.BlockSpec((D, D), lambda i: (0, 0)),
                pl.BlockSpec((D, 128), lambda i: (0, 0)),
                pl.BlockSpec((D, 384), lambda i: (0, 0)),
                pl.BlockSpec((D, 192), lambda i: (0, 0)),
            ],
            out_specs=[
                pl.BlockSpec((QB, NS_IDX), lambda i: (i, 0)),
                pl.BlockSpec((QB, NS_PER_ROW), lambda i: (i, 0)),
            ],
            out_shape=[
                jax.ShapeDtypeStruct((crows, NS_IDX), jnp.int32),
                jax.ShapeDtypeStruct((crows, NS_PER_ROW), jnp.float32),
            ],
        )(x2[row0:row0 + crows], Wq, wref_p, woff_p, wattn_p)

        sampled = pl.kernel(
            functools.partial(_sc_sample_body, crows // NW),
            mesh=mesh,
            compiler_params=pltpu.CompilerParams(use_tc_tiling_on_sc=False),
            out_type=jax.ShapeDtypeStruct((crows, D), jnp.float32),
            scratch_types=[
                pltpu.VMEM((3, NS_IDX), jnp.int32),
                pltpu.VMEM((3, NS_PER_ROW), jnp.float32),
                pltpu.VMEM((3, NS_IDX, 32), jnp.int32),
                pltpu.VMEM((D,), jnp.float32),
                pltpu.SemaphoreType.DMA((3,)),
                pltpu.SemaphoreType.DMA((3,)),
            ],
        )(table, idx, wgt)

        out = pl.pallas_call(
            _matmul_body,
            grid=(n_blocks,),
            in_specs=[
                pl.BlockSpec((QB, D), lambda i: (i, 0)),
                pl.BlockSpec((D, D), lambda i: (0, 0)),
            ],
            out_specs=pl.BlockSpec((QB, D), lambda i: (i, 0)),
            out_shape=jax.ShapeDtypeStruct((crows, D), jnp.float32),
        )(sampled, Wout)
        outs.append(out)

    return jnp.concatenate(outs, axis=0).reshape(B, LQ, D)


# speedup vs baseline: 1.3739x; 1.1792x over previous
import functools

import jax
import jax.numpy as jnp
from jax import lax
from jax.experimental import pallas as pl
from jax.experimental.pallas import tpu as pltpu
from jax.experimental.pallas import tpu_sc as plsc

B, LQ, D, H, L, P = 4, 2048, 512, 16, 3, 4
HD = D // H
LEVELS = ((64, 64, 0), (32, 32, 4096), (16, 16, 5120))
LV = 5376
QB = 256
NQR = B * LQ
NS_PER_ROW = H * L * P * 4
NC, NSUB, LANES = 2, 16, 16
NW = NC * NSUB
ROWS_PER_W = NQR // NW


def _head_body(row0, x_ref, wq_ref, wref_ref, woff_ref, wattn_ref,
               idx_ref, wgt_ref):
    xq = x_ref[...]
    q = jnp.dot(xq, wq_ref[...], preferred_element_type=jnp.float32)
    rr = jnp.dot(q, wref_ref[...], preferred_element_type=jnp.float32)
    off = jnp.dot(q, woff_ref[...], preferred_element_type=jnp.float32)
    al = jnp.dot(q, wattn_ref[...], preferred_element_type=jnp.float32)

    rx = jax.nn.sigmoid(rr[:, 0:64])
    ry = jax.nn.sigmoid(rr[:, 64:128])

    m16 = al[:, 0:16]
    for s in range(1, 12):
        m16 = jnp.maximum(m16, al[:, s * 16:(s + 1) * 16])
    e_ls = []
    s16 = jnp.zeros_like(m16)
    for lev in range(L):
        m64 = jnp.concatenate([m16, m16, m16, m16], axis=1)
        e = jnp.exp(al[:, lev * 64:(lev + 1) * 64] - m64)
        e_ls.append(e)
        for p_ in range(4):
            s16 = s16 + e[:, p_ * 16:(p_ + 1) * 16]
    inv16 = 1.0 / s16
    inv64 = jnp.concatenate([inv16, inv16, inv16, inv16], axis=1)

    h_lane = lax.broadcasted_iota(jnp.int32, (QB, 64), 1) % 16
    bidx = (row0 + pl.program_id(0) * QB) // LQ

    for lev, (hl, wl, start) in enumerate(LEVELS):
        attn_l = e_ls[lev] * inv64
        ox = off[:, (lev * 2 + 0) * 64:(lev * 2 + 1) * 64]
        oy = off[:, (lev * 2 + 1) * 64:(lev * 2 + 2) * 64]
        ix = (rx + ox) * wl - 0.5
        iy = (ry + oy) * hl - 0.5
        x0 = jnp.floor(ix)
        y0 = jnp.floor(iy)
        fx = ix - x0
        fy = iy - y0
        vx0 = ((x0 >= 0) & (x0 <= wl - 1)).astype(jnp.float32)
        vx1 = ((x0 >= -1) & (x0 <= wl - 2)).astype(jnp.float32)
        vy0 = ((y0 >= 0) & (y0 <= hl - 1)).astype(jnp.float32)
        vy1 = ((y0 >= -1) & (y0 <= hl - 2)).astype(jnp.float32)
        cx0 = jnp.clip(x0, 0, wl - 1).astype(jnp.int32)
        cx1 = jnp.clip(x0 + 1, 0, wl - 1).astype(jnp.int32)
        cy0 = jnp.clip(y0, 0, hl - 1).astype(jnp.int32)
        cy1 = jnp.clip(y0 + 1, 0, hl - 1).astype(jnp.int32)
        base = (bidx * LV + start) * 16
        wx = (1.0 - fx, fx)
        wy = (1.0 - fy, fy)
        cxs = (cx0, cx1)
        cys = (cy0, cy1)
        vxs = (vx0, vx1)
        vys = (vy0, vy1)
        for corner in range(4):
            yc, xc = corner // 2, corner % 2
            g = base + (cys[yc] * wl + cxs[xc]) * 16 + h_lane
            w = attn_l * (wx[xc] * wy[yc]) * (vxs[xc] * vys[yc])
            dst = lev * 256 + corner * 64
            idx_ref[:, pl.ds(dst, 64)] = g
            wgt_ref[:, pl.ds(dst, 64)] = w


def _matmul_body(a_ref, w_ref, o_ref):
    o_ref[...] = jnp.dot(a_ref[...], w_ref[...],
                         preferred_element_type=jnp.float32
                         ).astype(o_ref.dtype)


def _values_body(a_ref, w_ref, o_ref):
    o = jnp.dot(a_ref[...], w_ref[...], preferred_element_type=jnp.float32)
    lo = jnp.concatenate([o[:, h * HD:h * HD + 16] for h in range(H)], axis=1)
    hi = jnp.concatenate([o[:, h * HD + 16:(h + 1) * HD] for h in range(H)],
                         axis=1)
    lo_b = lax.bitcast_convert_type(lo.astype(jnp.bfloat16),
                                    jnp.uint16).astype(jnp.uint32)
    hi_b = lax.bitcast_convert_type(hi.astype(jnp.bfloat16),
                                    jnp.uint16).astype(jnp.uint32)
    o_ref[...] = lax.bitcast_convert_type(
        lo_b | lax.shift_left(hi_b, jnp.uint32(16)), jnp.int32)


def _splat(vec, h):
    idx = jnp.zeros((LANES,), jnp.int32) + h
    return lax.gather(
        vec, idx[:, None],
        lax.GatherDimensionNumbers(
            offset_dims=(), collapsed_slice_dims=(0,), start_index_map=(0,)),
        slice_sizes=(1,),
        mode=lax.GatherScatterMode.PROMISE_IN_BOUNDS)


def _sc_sample_body(rows_per_w, table_hbm, idx_hbm, wgt_hbm, out_hbm,
                    idx_v, wgt_v, rows_v, out_v, gsem, iwsem):
    wid = lax.axis_index("s") * NC + lax.axis_index("c")
    base = wid * rows_per_w
    nj = L * P * 4

    def issue_iw(r, slot):
        pltpu.async_copy(idx_hbm.at[base + r], idx_v.at[slot], iwsem.at[slot])
        pltpu.async_copy(wgt_hbm.at[base + r], wgt_v.at[slot], iwsem.at[slot])

    def wait_iw(slot):
        pltpu.make_async_copy(idx_hbm.at[0], idx_v.at[slot],
                              iwsem.at[slot]).wait()
        pltpu.make_async_copy(wgt_hbm.at[0], wgt_v.at[slot],
                              iwsem.at[slot]).wait()

    def issue_gathers(slot):
        pltpu.async_copy(table_hbm.at[idx_v.at[slot]],
                         rows_v.at[slot], gsem.at[slot])

    def wait_gathers(slot):
        pltpu.make_async_copy(table_hbm.at[pl.ds(0, NS_PER_ROW)],
                              rows_v.at[slot], gsem.at[slot]).wait()

    issue_iw(0, 0)
    issue_iw(1, 1)
    wait_iw(0)
    issue_gathers(0)

    def row_body(r, carry):
        slot = r % 3
        wait_gathers(slot)

        @pl.when(r + 2 < rows_per_w)
        def _():
            issue_iw(r + 2, (r + 2) % 3)

        @pl.when(r + 1 < rows_per_w)
        def _():
            nslot = (r + 1) % 3
            wait_iw(nslot)
            issue_gathers(nslot)

        for hbase in (0, 8):
            def j_body(j, accs, hbase=hbase):
                wv16 = wgt_v[slot, pl.ds(j * 16, 16)]
                new = []
                for hh in range(8):
                    h = hbase + hh
                    wsp = _splat(wv16, h)
                    packed = rows_v[slot, j * 16 + h]
                    va = lax.bitcast_convert_type(
                        lax.shift_left(packed, 16), jnp.float32)
                    vb = lax.bitcast_convert_type(packed, jnp.float32)
                    new.append(accs[2 * hh] + wsp * va)
                    new.append(accs[2 * hh + 1] + wsp * vb)
                return tuple(new)

            accs = lax.fori_loop(
                0, nj, j_body,
                tuple(jnp.zeros((LANES,), jnp.float32) for _ in range(16)))
            for hh in range(8):
                h = hbase + hh
                out_v[pl.ds(h * HD, 16)] = accs[2 * hh]
                out_v[pl.ds(h * HD + 16, 16)] = accs[2 * hh + 1]
        pltpu.sync_copy(out_v, out_hbm.at[base + r])
        return carry

    lax.fori_loop(0, rows_per_w, row_body, 0)


def kernel(x, encoder_input, spatial_shapes, Wq, Wref, Woff, Wattn, Wv, Wout):
    del spatial_shapes
    x2 = x.reshape(NQR, D)
    enc2 = encoder_input.reshape(B * LV, D)

    woff_p = Woff.reshape(D, H, L, P, 2).transpose(0, 2, 4, 3, 1).reshape(D, 384)
    wattn_p = Wattn.reshape(D, H, L * P).transpose(0, 2, 1).reshape(D, 192)
    wref_p = jnp.concatenate([jnp.tile(Wref[:, 0:1], (1, 64)),
                              jnp.tile(Wref[:, 1:2], (1, 64))], axis=1)

    vt = pl.pallas_call(
        _values_body,
        grid=(B * LV // QB,),
        in_specs=[
            pl.BlockSpec((QB, D), lambda i: (i, 0)),
            pl.BlockSpec((D, D), lambda i: (0, 0)),
        ],
        out_specs=pl.BlockSpec((QB, D // 2), lambda i: (i, 0)),
        out_shape=jax.ShapeDtypeStruct((B * LV, D // 2), jnp.int32),
    )(enc2, Wv)

    table = vt.reshape(B * LV * H, 16)
    mesh = plsc.VectorSubcoreMesh(core_axis_name="c", subcore_axis_name="s")

    nchunk = 4
    crows = NQR // nchunk
    n_blocks = crows // QB
    outs = []
    for ci in range(nchunk):
        row0 = ci * crows
        idx, wgt = pl.pallas_call(
            functools.partial(_head_body, row0),
            grid=(n_blocks,),
            in_specs=[
                pl.BlockSpec((QB, D), lambda i: (i, 0)),
                pl.BlockSpec((D, D), lambda i: (0, 0)),
                pl.BlockSpec((D, 128), lambda i: (0, 0)),
                pl.BlockSpec((D, 384), lambda i: (0, 0)),
                pl.BlockSpec((D, 192), lambda i: (0, 0)),
            ],
            out_specs=[
                pl.BlockSpec((QB, NS_PER_ROW), lambda i: (i, 0)),
                pl.BlockSpec((QB, NS_PER_ROW), lambda i: (i, 0)),
            ],
            out_shape=[
                jax.ShapeDtypeStruct((crows, NS_PER_ROW), jnp.int32),
                jax.ShapeDtypeStruct((crows, NS_PER_ROW), jnp.float32),
            ],
        )(x2[row0:row0 + crows], Wq, wref_p, woff_p, wattn_p)

        sampled = pl.kernel(
            functools.partial(_sc_sample_body, crows // NW),
            mesh=mesh,
            compiler_params=pltpu.CompilerParams(use_tc_tiling_on_sc=False),
            out_type=jax.ShapeDtypeStruct((crows, D), jnp.float32),
            scratch_types=[
                pltpu.VMEM((3, NS_PER_ROW), jnp.int32),
                pltpu.VMEM((3, NS_PER_ROW), jnp.float32),
                pltpu.VMEM((3, NS_PER_ROW, 16), jnp.int32),
                pltpu.VMEM((D,), jnp.float32),
                pltpu.SemaphoreType.DMA((3,)),
                pltpu.SemaphoreType.DMA((3,)),
            ],
        )(table, idx, wgt)

        out = pl.pallas_call(
            _matmul_body,
            grid=(n_blocks,),
            in_specs=[
                pl.BlockSpec((QB, D), lambda i: (i, 0)),
                pl.BlockSpec((D, D), lambda i: (0, 0)),
            ],
            out_specs=pl.BlockSpec((QB, D), lambda i: (i, 0)),
            out_shape=jax.ShapeDtypeStruct((crows, D), jnp.float32),
        )(sampled, Wout)
        outs.append(out)

    return jnp.concatenate(outs, axis=0).reshape(B, LQ, D)
